# Initial kernel scaffold; baseline (speedup 1.0000x reference)
import jax
import jax.numpy as jnp
from jax.experimental import pallas as pl


def kernel(x, edge_index, edge_attr, params):
    raise NotImplementedError("write your pallas kernel here")


# speedup vs baseline: 14.1636x; 14.1636x over previous
import functools

import jax
import jax.numpy as jnp
from jax import lax
from jax.experimental import pallas as pl
from jax.experimental.pallas import tpu as pltpu
from jax.experimental.pallas import tpu_sc as plsc

N_NODES = 10000
E_EDGES = 320000
D = 128
ED = 16
N_LAYERS_K = 4
OUT_DIM_K = 256

W_SC = 128
NPAD = 10240
ROWS_PER_SUBCORE = NPAD // 16
PACK = E_EDGES // 8
HB = N_NODES // 8
NPK = NPAD // 8

_SQRT2 = 1.4142135623730951


def _gelu(x):
    return x * 0.5 * (1.0 + lax.erf(x / _SQRT2))


def _proj_body(hb_ref, wi_ref, wj_ref, a_ref, b_ref):
    hb = hb_ref[...]
    a_ref[...] = jnp.dot(hb, wi_ref[...], preferred_element_type=jnp.float32)
    b_ref[...] = jnp.dot(hb, wj_ref[...], preferred_element_type=jnp.float32)


def _proj(hb, wgi, wgj):
    return pl.pallas_call(
        _proj_body,
        out_shape=[jax.ShapeDtypeStruct((HB, D), jnp.float32)] * 2,
    )(hb, wgi, wgj)


def _edge_mlp_body(ga_ref, gb_ref, ea_ref, w1_ref, b1_ref, w2_ref, b2_ref, m_ref):
    x = (
        ga_ref[...]
        + gb_ref[...]
        + jnp.dot(ea_ref[...], w1_ref[...], preferred_element_type=jnp.float32)
        + b1_ref[...]
    )
    t = _gelu(x)
    m_ref[...] = _gelu(
        jnp.dot(t, w2_ref[...], preferred_element_type=jnp.float32) + b2_ref[...]
    )


def _edge_mlp(ga_v, gb_v, ea_v, w1e_blk, b1t, w2_blk, b2t):
    blk = 2000
    return pl.pallas_call(
        _edge_mlp_body,
        grid=(PACK // blk,),
        in_specs=[
            pl.BlockSpec((blk, D), lambda i: (i, 0)),
            pl.BlockSpec((blk, D), lambda i: (i, 0)),
            pl.BlockSpec((blk, D), lambda i: (i, 0)),
            pl.BlockSpec((D, D), lambda i: (0, 0)),
            pl.BlockSpec((1, D), lambda i: (0, 0)),
            pl.BlockSpec((D, D), lambda i: (0, 0)),
            pl.BlockSpec((1, D), lambda i: (0, 0)),
        ],
        out_specs=pl.BlockSpec((blk, D), lambda i: (i, 0)),
        out_shape=jax.ShapeDtypeStruct((PACK, D), jnp.float32),
    )(ga_v, gb_v, ea_v, w1e_blk, b1t, w2_blk, b2t)


def _aggr_prep_body(p_ref, wu_ref, o_ref):
    s = p_ref[0] + p_ref[1]
    o_ref[...] = jnp.dot(s, wu_ref[...], preferred_element_type=jnp.float32)


def _aggr_prep(parts_pk, wu):
    return pl.pallas_call(
        _aggr_prep_body,
        out_shape=jax.ShapeDtypeStruct((NPK, 8 * D), jnp.float32),
    )(parts_pk, wu)


def _update_body(h_ref, ua_ref, w1h_ref, b1_ref, w2_ref, b2_ref, g_ref, bb_ref, hn_ref):
    h = h_ref[...]
    u = _gelu(
        jnp.dot(h, w1h_ref[...], preferred_element_type=jnp.float32)
        + ua_ref[...]
        + b1_ref[...]
    )
    u = _gelu(jnp.dot(u, w2_ref[...], preferred_element_type=jnp.float32) + b2_ref[...])
    o = h + u
    mean = jnp.mean(o, axis=1, keepdims=True)
    c = o - mean
    var = jnp.mean(c * c, axis=1, keepdims=True)
    hn_ref[...] = c * lax.rsqrt(var + 1e-5) * g_ref[...] + bb_ref[...]


def _update(h, ua, w1h, b1, w2, b2, g, bb):
    blk = 2000
    return pl.pallas_call(
        _update_body,
        grid=(N_NODES // blk,),
        in_specs=[
            pl.BlockSpec((blk, D), lambda i: (i, 0)),
            pl.BlockSpec((blk, D), lambda i: (i, 0)),
            pl.BlockSpec((D, D), lambda i: (0, 0)),
            pl.BlockSpec((1, D), lambda i: (0, 0)),
            pl.BlockSpec((D, D), lambda i: (0, 0)),
            pl.BlockSpec((1, D), lambda i: (0, 0)),
            pl.BlockSpec((1, D), lambda i: (0, 0)),
            pl.BlockSpec((1, D), lambda i: (0, 0)),
        ],
        out_specs=pl.BlockSpec((blk, D), lambda i: (i, 0)),
        out_shape=jax.ShapeDtypeStruct((N_NODES, D), jnp.float32),
    )(h, ua, w1h, b1, w2, b2, g, bb)


def _readout_body(h_ref, w_ref, b_ref, o_ref):
    gmean = jnp.mean(h_ref[...], axis=0, keepdims=True)
    o_ref[...] = _gelu(
        jnp.dot(gmean, w_ref[...], preferred_element_type=jnp.float32) + b_ref[...]
    )


def _readout(h, proj_w, proj_b):
    return pl.pallas_call(
        _readout_body,
        out_shape=jax.ShapeDtypeStruct((1, OUT_DIM_K), jnp.float32),
    )(h, proj_w, proj_b)


_SC_PARAMS = pltpu.CompilerParams(use_tc_tiling_on_sc=False)


def _sc_mesh():
    return plsc.VectorSubcoreMesh(core_axis_name="core", subcore_axis_name="subcore")


def _sc_gather(a_tab, b_tab, dst2d, src2d):
    @functools.partial(
        pl.kernel,
        out_type=[jax.ShapeDtypeStruct((E_EDGES, ED), jnp.float32)] * 2,
        mesh=_sc_mesh(),
        compiler_params=_SC_PARAMS,
    )
    def k(a_hbm, b_hbm, d_hbm, s_hbm, ga_hbm, gb_hbm):
        def body(d_vmem, s_vmem, ga_vmem, gb_vmem):
            pltpu.sync_copy(a_hbm.at[d_vmem.at[0]], ga_vmem)
            pltpu.sync_copy(b_hbm.at[s_vmem.at[0]], gb_vmem)

        pltpu.emit_pipeline(
            body,
            grid=(E_EDGES // W_SC,),
            in_specs=[
                pl.BlockSpec((1, W_SC), lambda i: (0, i)),
                pl.BlockSpec((1, W_SC), lambda i: (0, i)),
            ],
            out_specs=[
                pl.BlockSpec((W_SC, ED), lambda i: (i, 0)),
                pl.BlockSpec((W_SC, ED), lambda i: (i, 0)),
            ],
            core_axis_name=("core", "subcore"),
            dimension_semantics=(pltpu.PARALLEL,),
        )(d_hbm, s_hbm, ga_hbm, gb_hbm)

    return k(a_tab, b_tab, dst2d, src2d)


def _sc_scatter(m, dst2d):
    @functools.partial(
        pl.kernel,
        out_type=jax.ShapeDtypeStruct((2, NPAD, ED), jnp.float32),
        mesh=_sc_mesh(),
        compiler_params=_SC_PARAMS,
        scratch_types=[
            pltpu.VMEM_SHARED((NPAD, ED), jnp.float32),
            pltpu.VMEM((ROWS_PER_SUBCORE, ED), jnp.float32),
        ],
    )
    def k(m_hbm, d_hbm, o_hbm, acc, zbuf):
        cid = lax.axis_index("core")
        sid = lax.axis_index("subcore")
        row0 = sid * ROWS_PER_SUBCORE

        @pl.loop(0, ROWS_PER_SUBCORE)
        def _(i):
            zbuf[i] = jnp.zeros((ED,), jnp.float32)

        pltpu.sync_copy(zbuf, acc.at[pl.ds(row0, ROWS_PER_SUBCORE)])
        plsc.subcore_barrier()

        def body(m_vmem, d_vmem):
            pltpu.sync_copy(m_vmem, acc.at[d_vmem.at[0]], add=True)

        pltpu.emit_pipeline(
            body,
            grid=(E_EDGES // W_SC,),
            in_specs=[
                pl.BlockSpec((W_SC, ED), lambda i: (i, 0)),
                pl.BlockSpec((1, W_SC), lambda i: (0, i)),
            ],
            out_specs=[],
            core_axis_name=("core", "subcore"),
            dimension_semantics=(pltpu.PARALLEL,),
        )(m_hbm, d_hbm)

        plsc.subcore_barrier()
        pltpu.sync_copy(
            acc.at[pl.ds(row0, ROWS_PER_SUBCORE)],
            o_hbm.at[cid].at[pl.ds(row0, ROWS_PER_SUBCORE)],
        )

    return k(m, dst2d)


def kernel(x, edge_index, edge_attr, params):
    src2d = edge_index[0].reshape(1, E_EDGES)
    dst2d = edge_index[1].reshape(1, E_EDGES)
    layers = params["layers"]
    eye8 = jnp.eye(8, dtype=jnp.float32)
    ea_v = edge_attr.reshape(PACK, D)

    h = x
    for p in layers:
        wgi = jnp.kron(eye8, p["mW1"][:D])
        wgj = jnp.kron(eye8, p["mW1"][D : 2 * D])
        w1e_blk = jnp.kron(eye8, p["mW1"][2 * D :])
        w2_blk = jnp.kron(eye8, p["mW2"])
        b1t = jnp.tile(p["mb1"], 8).reshape(1, D)
        b2t = jnp.tile(p["mb2"], 8).reshape(1, D)
        wu = jnp.kron(eye8, p["uW1"][D:])

        a_pk, b_pk = _proj(h.reshape(HB, 8 * D), wgi, wgj)
        ga, gb = _sc_gather(
            a_pk.reshape(N_NODES, ED), b_pk.reshape(N_NODES, ED), dst2d, src2d
        )
        m = _edge_mlp(
            ga.reshape(PACK, D), gb.reshape(PACK, D), ea_v,
            w1e_blk, b1t, w2_blk, b2t,
        )
        parts = _sc_scatter(m.reshape(E_EDGES, ED), dst2d)
        ua = _aggr_prep(parts.reshape(2, NPK, D), wu)
        h = _update(
            h, ua.reshape(NPAD, D),
            p["uW1"][:D], p["ub1"].reshape(1, D),
            p["uW2"], p["ub2"].reshape(1, D),
            p["g"].reshape(1, D), p["b"].reshape(1, D),
        )

    return _readout(h, params["projW"], params["projb"].reshape(1, OUT_DIM_K))


# speedup vs baseline: 16.2028x; 1.1440x over previous
import functools

import jax
import jax.numpy as jnp
from jax import lax
from jax.experimental import pallas as pl
from jax.experimental.pallas import tpu as pltpu
from jax.experimental.pallas import tpu_sc as plsc

N_NODES = 10000
E_EDGES = 320000
D = 128
ED = 16
N_LAYERS_K = 4
OUT_DIM_K = 256

W_SC = 128
K_SC = 10
STEP = W_SC * K_SC
N_WIN = E_EDGES // W_SC
NPAD = 10240
ROWS_PER_SUBCORE = NPAD // 16
PACK = E_EDGES // 8
HB = N_NODES // 8
NPK = NPAD // 8

_SQRT2 = 1.4142135623730951


def _gelu(x):
    return x * 0.5 * (1.0 + lax.erf(x / _SQRT2))


def _proj_body(hb_ref, wi_ref, wj_ref, a_ref, b_ref):
    hb = hb_ref[...]
    a_ref[...] = jnp.dot(hb, wi_ref[...], preferred_element_type=jnp.float32)
    b_ref[...] = jnp.dot(hb, wj_ref[...], preferred_element_type=jnp.float32)


def _proj(hb, wgi, wgj):
    return pl.pallas_call(
        _proj_body,
        out_shape=[jax.ShapeDtypeStruct((HB, D), jnp.float32)] * 2,
    )(hb, wgi, wgj)


def _edge_mlp_body(ga_ref, gb_ref, ea_ref, w1_ref, b1_ref, w2_ref, b2_ref, m_ref):
    x = (
        ga_ref[...]
        + gb_ref[...]
        + jnp.dot(ea_ref[...], w1_ref[...], preferred_element_type=jnp.float32)
        + b1_ref[...]
    )
    t = _gelu(x)
    m_ref[...] = _gelu(
        jnp.dot(t, w2_ref[...], preferred_element_type=jnp.float32) + b2_ref[...]
    )


def _edge_mlp(ga_v, gb_v, ea_v, w1e_blk, b1t, w2_blk, b2t):
    blk = 2000
    return pl.pallas_call(
        _edge_mlp_body,
        grid=(PACK // blk,),
        in_specs=[
            pl.BlockSpec((blk, D), lambda i: (i, 0)),
            pl.BlockSpec((blk, D), lambda i: (i, 0)),
            pl.BlockSpec((blk, D), lambda i: (i, 0)),
            pl.BlockSpec((D, D), lambda i: (0, 0)),
            pl.BlockSpec((1, D), lambda i: (0, 0)),
            pl.BlockSpec((D, D), lambda i: (0, 0)),
            pl.BlockSpec((1, D), lambda i: (0, 0)),
        ],
        out_specs=pl.BlockSpec((blk, D), lambda i: (i, 0)),
        out_shape=jax.ShapeDtypeStruct((PACK, D), jnp.float32),
    )(ga_v, gb_v, ea_v, w1e_blk, b1t, w2_blk, b2t)


def _aggr_prep_body(p_ref, wu_ref, o_ref):
    s = p_ref[0] + p_ref[1]
    o_ref[...] = jnp.dot(s, wu_ref[...], preferred_element_type=jnp.float32)


def _aggr_prep(parts_pk, wu):
    return pl.pallas_call(
        _aggr_prep_body,
        out_shape=jax.ShapeDtypeStruct((NPK, 8 * D), jnp.float32),
    )(parts_pk, wu)


def _update_body(h_ref, ua_ref, w1h_ref, b1_ref, w2_ref, b2_ref, g_ref, bb_ref, hn_ref):
    h = h_ref[...]
    u = _gelu(
        jnp.dot(h, w1h_ref[...], preferred_element_type=jnp.float32)
        + ua_ref[...]
        + b1_ref[...]
    )
    u = _gelu(jnp.dot(u, w2_ref[...], preferred_element_type=jnp.float32) + b2_ref[...])
    o = h + u
    mean = jnp.mean(o, axis=1, keepdims=True)
    c = o - mean
    var = jnp.mean(c * c, axis=1, keepdims=True)
    hn_ref[...] = c * lax.rsqrt(var + 1e-5) * g_ref[...] + bb_ref[...]


def _update(h, ua, w1h, b1, w2, b2, g, bb):
    blk = 2000
    return pl.pallas_call(
        _update_body,
        grid=(N_NODES // blk,),
        in_specs=[
            pl.BlockSpec((blk, D), lambda i: (i, 0)),
            pl.BlockSpec((blk, D), lambda i: (i, 0)),
            pl.BlockSpec((D, D), lambda i: (0, 0)),
            pl.BlockSpec((1, D), lambda i: (0, 0)),
            pl.BlockSpec((D, D), lambda i: (0, 0)),
            pl.BlockSpec((1, D), lambda i: (0, 0)),
            pl.BlockSpec((1, D), lambda i: (0, 0)),
            pl.BlockSpec((1, D), lambda i: (0, 0)),
        ],
        out_specs=pl.BlockSpec((blk, D), lambda i: (i, 0)),
        out_shape=jax.ShapeDtypeStruct((N_NODES, D), jnp.float32),
    )(h, ua, w1h, b1, w2, b2, g, bb)


def _readout_body(h_ref, w_ref, b_ref, o_ref):
    gmean = jnp.mean(h_ref[...], axis=0, keepdims=True)
    o_ref[...] = _gelu(
        jnp.dot(gmean, w_ref[...], preferred_element_type=jnp.float32) + b_ref[...]
    )


def _readout(h, proj_w, proj_b):
    return pl.pallas_call(
        _readout_body,
        out_shape=jax.ShapeDtypeStruct((1, OUT_DIM_K), jnp.float32),
    )(h, proj_w, proj_b)


_SC_PARAMS = pltpu.CompilerParams(use_tc_tiling_on_sc=False)


def _sc_mesh():
    return plsc.VectorSubcoreMesh(core_axis_name="core", subcore_axis_name="subcore")


def _sc_gather(a_tab, b_tab, dst2d, src2d):
    @functools.partial(
        pl.kernel,
        out_type=[jax.ShapeDtypeStruct((E_EDGES, ED), jnp.float32)] * 2,
        mesh=_sc_mesh(),
        compiler_params=_SC_PARAMS,
        scratch_types=[pltpu.SemaphoreType.DMA, pltpu.SemaphoreType.DMA],
    )
    def k(a_hbm, b_hbm, d_hbm, s_hbm, ga_hbm, gb_hbm, sema, semb):
        def body(d_vmem, s_vmem, ga_vmem, gb_vmem):
            copies = []
            for j in range(K_SC):
                sl = pl.ds(j * W_SC, W_SC)
                copies.append(
                    pltpu.async_copy(a_hbm.at[d_vmem.at[j]], ga_vmem.at[sl], sema)
                )
                copies.append(
                    pltpu.async_copy(b_hbm.at[s_vmem.at[j]], gb_vmem.at[sl], semb)
                )
            for c in copies:
                c.wait()

        pltpu.emit_pipeline(
            body,
            grid=(E_EDGES // STEP,),
            in_specs=[
                pl.BlockSpec((K_SC, W_SC), lambda i: (i, 0)),
                pl.BlockSpec((K_SC, W_SC), lambda i: (i, 0)),
            ],
            out_specs=[
                pl.BlockSpec((STEP, ED), lambda i: (i, 0)),
                pl.BlockSpec((STEP, ED), lambda i: (i, 0)),
            ],
            core_axis_name=("core", "subcore"),
            dimension_semantics=(pltpu.PARALLEL,),
        )(d_hbm, s_hbm, ga_hbm, gb_hbm)

    return k(a_tab, b_tab, dst2d, src2d)


def _sc_scatter(m, dst2d):
    @functools.partial(
        pl.kernel,
        out_type=jax.ShapeDtypeStruct((2, NPAD, ED), jnp.float32),
        mesh=_sc_mesh(),
        compiler_params=_SC_PARAMS,
        scratch_types=[
            pltpu.VMEM_SHARED((NPAD, ED), jnp.float32),
            pltpu.VMEM((ROWS_PER_SUBCORE, ED), jnp.float32),
            pltpu.SemaphoreType.DMA,
        ],
    )
    def k(m_hbm, d_hbm, o_hbm, acc, zbuf, sem):
        cid = lax.axis_index("core")
        sid = lax.axis_index("subcore")
        row0 = sid * ROWS_PER_SUBCORE

        @pl.loop(0, ROWS_PER_SUBCORE)
        def _(i):
            zbuf[i] = jnp.zeros((ED,), jnp.float32)

        pltpu.sync_copy(zbuf, acc.at[pl.ds(row0, ROWS_PER_SUBCORE)])
        plsc.subcore_barrier()

        def body(m_vmem, d_vmem):
            copies = []
            for j in range(K_SC):
                sl = pl.ds(j * W_SC, W_SC)
                copies.append(
                    pltpu.async_copy(m_vmem.at[sl], acc.at[d_vmem.at[j]], sem, add=True)
                )
            for c in copies:
                c.wait()

        pltpu.emit_pipeline(
            body,
            grid=(E_EDGES // STEP,),
            in_specs=[
                pl.BlockSpec((STEP, ED), lambda i: (i, 0)),
                pl.BlockSpec((K_SC, W_SC), lambda i: (i, 0)),
            ],
            out_specs=[],
            core_axis_name=("core", "subcore"),
            dimension_semantics=(pltpu.PARALLEL,),
        )(m_hbm, d_hbm)

        plsc.subcore_barrier()
        pltpu.sync_copy(
            acc.at[pl.ds(row0, ROWS_PER_SUBCORE)],
            o_hbm.at[cid].at[pl.ds(row0, ROWS_PER_SUBCORE)],
        )

    return k(m, dst2d)


def kernel(x, edge_index, edge_attr, params):
    src2d = edge_index[0].reshape(N_WIN, W_SC)
    dst2d = edge_index[1].reshape(N_WIN, W_SC)
    layers = params["layers"]
    eye8 = jnp.eye(8, dtype=jnp.float32)
    ea_v = edge_attr.reshape(PACK, D)

    h = x
    for p in layers:
        wgi = jnp.kron(eye8, p["mW1"][:D])
        wgj = jnp.kron(eye8, p["mW1"][D : 2 * D])
        w1e_blk = jnp.kron(eye8, p["mW1"][2 * D :])
        w2_blk = jnp.kron(eye8, p["mW2"])
        b1t = jnp.tile(p["mb1"], 8).reshape(1, D)
        b2t = jnp.tile(p["mb2"], 8).reshape(1, D)
        wu = jnp.kron(eye8, p["uW1"][D:])

        a_pk, b_pk = _proj(h.reshape(HB, 8 * D), wgi, wgj)
        ga, gb = _sc_gather(
            a_pk.reshape(N_NODES, ED), b_pk.reshape(N_NODES, ED), dst2d, src2d
        )
        m = _edge_mlp(
            ga.reshape(PACK, D), gb.reshape(PACK, D), ea_v,
            w1e_blk, b1t, w2_blk, b2t,
        )
        parts = _sc_scatter(m.reshape(E_EDGES, ED), dst2d)
        ua = _aggr_prep(parts.reshape(2, NPK, D), wu)
        h = _update(
            h, ua.reshape(NPAD, D),
            p["uW1"][:D], p["ub1"].reshape(1, D),
            p["uW2"], p["ub2"].reshape(1, D),
            p["g"].reshape(1, D), p["b"].reshape(1, D),
        )

    return _readout(h, params["projW"], params["projb"].reshape(1, OUT_DIM_K))
